# baseline (speedup 1.0000x reference)
import functools

import jax
import jax.numpy as jnp
from jax import lax
from jax.experimental import pallas as pl
from jax.experimental.pallas import tpu as pltpu
from jax.experimental.pallas import tpu_sc as plsc

N = 10000
D = 128
E = 320000

NC = 2
NS = 16
CHUNK = 128
BLKI = 8
NBLK = 10
CPT = NBLK * BLKI
E_PAD = NC * NS * CPT * CHUNK
N_PAD = 10240
RPT = N_PAD // NS

_MESH = dict(core_axis_name="c", subcore_axis_name="s")


def _sc_deg(dsts, ones1, zeros1):
    @functools.partial(
        pl.kernel,
        mesh=plsc.VectorSubcoreMesh(**_MESH),
        out_type=jax.ShapeDtypeStruct((NC, N_PAD), jnp.float32),
        scratch_types=[
            pltpu.VMEM((NBLK, BLKI, CHUNK), jnp.int32),
            pltpu.VMEM((CHUNK,), jnp.float32),
            pltpu.VMEM_SHARED((N_PAD,), jnp.float32),
        ],
    )
    def k(dsts_hbm, ones_hbm, zeros_hbm, out_hbm, dst_v, ones_v, acc_sh):
        c = lax.axis_index("c")
        s = lax.axis_index("s")
        rows = pl.ds(s * RPT, RPT)
        pltpu.sync_copy(dsts_hbm.at[c, s], dst_v)
        pltpu.sync_copy(ones_hbm, ones_v)
        pltpu.sync_copy(zeros_hbm.at[rows], acc_sh.at[rows])
        plsc.subcore_barrier()

        def body(B, carry):
            for r in range(BLKI):
                pltpu.sync_copy(ones_v, acc_sh.at[dst_v.at[B, r]], add=True)
            return carry

        lax.fori_loop(0, NBLK, body, 0)
        plsc.subcore_barrier()
        pltpu.sync_copy(acc_sh.at[rows], out_hbm.at[c, rows])

    return k(dsts, ones1, zeros1)


def _sc_prop(u, srcs, dsts, zeros):
    @functools.partial(
        pl.kernel,
        mesh=plsc.VectorSubcoreMesh(**_MESH),
        out_type=jax.ShapeDtypeStruct((NC, N_PAD, D), jnp.float32),
        scratch_types=[
            pltpu.VMEM((2, BLKI, CHUNK), jnp.int32),
            pltpu.VMEM((2, BLKI, CHUNK), jnp.int32),
            pltpu.VMEM((2, CHUNK, D), jnp.float32),
            pltpu.VMEM_SHARED((N_PAD, D), jnp.float32),
            pltpu.SemaphoreType.DMA,
            pltpu.SemaphoreType.DMA,
            pltpu.SemaphoreType.DMA,
            pltpu.SemaphoreType.DMA,
            pltpu.SemaphoreType.DMA,
        ],
    )
    def k(u_hbm, srcs_hbm, dsts_hbm, z_hbm, out_hbm,
          sidx, didx, buf_v, acc_sh, sg0, sg1, ss0, ss1, si):
        c = lax.axis_index("c")
        s = lax.axis_index("s")
        rows = pl.ds(s * RPT, RPT)
        sgs = (sg0, sg1)
        sss = (ss0, ss1)

        pltpu.async_copy(srcs_hbm.at[c, s, 0], sidx.at[0], si)
        pltpu.async_copy(dsts_hbm.at[c, s, 0], didx.at[0], si)

        @pl.when(c == 0)
        def _():
            pltpu.sync_copy(u_hbm.at[rows], acc_sh.at[rows])

        @pl.when(c != 0)
        def _():
            pltpu.sync_copy(z_hbm.at[rows], acc_sh.at[rows])

        pltpu.make_async_copy(srcs_hbm.at[c, s, 0], sidx.at[0], si).wait()
        pltpu.make_async_copy(dsts_hbm.at[c, s, 0], didx.at[0], si).wait()
        plsc.subcore_barrier()

        pltpu.async_copy(u_hbm.at[sidx.at[0, 0]], buf_v.at[0], sg0)

        def chunk(B, pb, r, first=False, last_block=False):
            par = r % 2
            pltpu.make_async_copy(
                u_hbm.at[sidx.at[pb, r]], buf_v.at[par], sgs[par]).wait()
            pltpu.async_copy(
                buf_v.at[par], acc_sh.at[didx.at[pb, r]], sss[par], add=True)
            if not first:
                pltpu.make_async_copy(
                    buf_v.at[1 - par], acc_sh.at[didx.at[pb, r]],
                    sss[1 - par]).wait()
            if r < BLKI - 1:
                pltpu.async_copy(
                    u_hbm.at[sidx.at[pb, r + 1]], buf_v.at[1 - par], sgs[1 - par])
            elif not last_block:
                pltpu.async_copy(
                    u_hbm.at[sidx.at[1 - pb, 0]], buf_v.at[1 - par],
                    sgs[1 - par])
            if r == 0 and not last_block:
                pltpu.async_copy(srcs_hbm.at[c, s, B + 1], sidx.at[1 - pb], si)
                pltpu.async_copy(dsts_hbm.at[c, s, B + 1], didx.at[1 - pb], si)
            if r == BLKI - 2 and not last_block:
                pltpu.make_async_copy(
                    srcs_hbm.at[c, s, 0], sidx.at[1 - pb], si).wait()
                pltpu.make_async_copy(
                    dsts_hbm.at[c, s, 0], didx.at[1 - pb], si).wait()

        for r in range(BLKI):
            chunk(0, 0, r, first=(r == 0))

        def pair(i, carry):
            B = 1 + 2 * i
            for pb, dB in ((1, 0), (0, 1)):
                for r in range(BLKI):
                    chunk(B + dB, pb, r)
            return carry

        lax.fori_loop(0, (NBLK - 2) // 2, pair, 0)

        for r in range(BLKI):
            chunk(NBLK - 1, (NBLK - 1) % 2, r, last_block=True)

        pltpu.make_async_copy(
            buf_v.at[(BLKI - 1) % 2], acc_sh.at[didx.at[0, 0]],
            sss[(BLKI - 1) % 2]).wait()
        plsc.subcore_barrier()
        pltpu.sync_copy(acc_sh.at[rows], out_hbm.at[c, rows])

    return k(u, srcs, dsts, zeros)


_BLK = 2048


def _dinv_of(deg_ref):
    return lax.rsqrt(deg_ref[:, 0:1] + deg_ref[:, 1:2] + 1.0)


def _tc_first(x, W, deg):
    def body(x_ref, w_ref, deg_ref, u_ref):
        dinv = _dinv_of(deg_ref)
        u_ref[...] = dinv * jnp.dot(
            x_ref[...], w_ref[...], preferred_element_type=jnp.float32)

    return pl.pallas_call(
        body,
        grid=(N_PAD // _BLK,),
        in_specs=[
            pl.BlockSpec((_BLK, D), lambda i: (i, 0)),
            pl.BlockSpec((D, D), lambda i: (0, 0)),
            pl.BlockSpec((_BLK, NC), lambda i: (i, 0)),
        ],
        out_specs=pl.BlockSpec((_BLK, D), lambda i: (i, 0)),
        out_shape=jax.ShapeDtypeStruct((N_PAD, D), jnp.float32),
    )(x, W, deg)


def _tc_mid(part, b, W, deg):
    def body(p_ref, b_ref, w_ref, deg_ref, u_ref):
        dinv = _dinv_of(deg_ref)
        t = dinv * (p_ref[0] + p_ref[1]) + b_ref[...]
        t = jnp.maximum(t, 0.0)
        u_ref[...] = dinv * jnp.dot(
            t, w_ref[...], preferred_element_type=jnp.float32)

    return pl.pallas_call(
        body,
        grid=(N_PAD // _BLK,),
        in_specs=[
            pl.BlockSpec((NC, _BLK, D), lambda i: (0, i, 0)),
            pl.BlockSpec((1, D), lambda i: (0, 0)),
            pl.BlockSpec((D, D), lambda i: (0, 0)),
            pl.BlockSpec((_BLK, NC), lambda i: (i, 0)),
        ],
        out_specs=pl.BlockSpec((_BLK, D), lambda i: (i, 0)),
        out_shape=jax.ShapeDtypeStruct((N_PAD, D), jnp.float32),
    )(part, b, W, deg)


def _tc_last(part, b, deg):
    def body(p_ref, b_ref, deg_ref, o_ref):
        dinv = _dinv_of(deg_ref)
        o_ref[...] = dinv * (p_ref[0] + p_ref[1]) + b_ref[...]

    return pl.pallas_call(
        body,
        grid=(N_PAD // _BLK,),
        in_specs=[
            pl.BlockSpec((NC, _BLK, D), lambda i: (0, i, 0)),
            pl.BlockSpec((1, D), lambda i: (0, 0)),
            pl.BlockSpec((_BLK, NC), lambda i: (i, 0)),
        ],
        out_specs=pl.BlockSpec((_BLK, D), lambda i: (i, 0)),
        out_shape=jax.ShapeDtypeStruct((N_PAD, D), jnp.float32),
    )(part, b, deg)


def kernel(x, edge_index, W1, b1, W2, b2, W3, b3):
    src = edge_index[0].astype(jnp.int32)
    dst = edge_index[1].astype(jnp.int32)
    pad = N + jnp.arange(E_PAD - E, dtype=jnp.int32) % (N_PAD - N)
    srcs = jnp.concatenate([src, pad]).reshape(NC, NS, NBLK, BLKI, CHUNK)
    dsts = jnp.concatenate([dst, pad]).reshape(NC, NS, NBLK, BLKI, CHUNK)

    x_pad = jnp.zeros((N_PAD, D), jnp.float32).at[:N].set(x)
    zeros128 = jnp.zeros((N_PAD, D), jnp.float32)
    zeros1 = jnp.zeros((N_PAD,), jnp.float32)
    ones1 = jnp.ones((CHUNK,), jnp.float32)
    b1r = b1.reshape(1, D)
    b2r = b2.reshape(1, D)
    b3r = b3.reshape(1, D)

    deg = _sc_deg(dsts, ones1, zeros1).T
    u = _tc_first(x_pad, W1, deg)
    p = _sc_prop(u, srcs, dsts, zeros128)
    u = _tc_mid(p, b1r, W2, deg)
    p = _sc_prop(u, srcs, dsts, zeros128)
    u = _tc_mid(p, b2r, W3, deg)
    p = _sc_prop(u, srcs, dsts, zeros128)
    out = _tc_last(p, b3r, deg)
    return out[:N]


# speedup vs baseline: 1.1634x; 1.1634x over previous
import functools

import jax
import jax.numpy as jnp
from jax import lax
from jax.experimental import pallas as pl
from jax.experimental.pallas import tpu as pltpu
from jax.experimental.pallas import tpu_sc as plsc

N = 10000
D = 128
E = 320000

NC = 2
NS = 16
CHUNK = 128
BLKI = 8
NBLK = 10
CPT = NBLK * BLKI
E_PAD = NC * NS * CPT * CHUNK
N_PAD = 10240
RPT = N_PAD // NS

_MESH = dict(core_axis_name="c", subcore_axis_name="s")


def _sc_deg(dsts, ones1, zeros1):
    @functools.partial(
        pl.kernel,
        mesh=plsc.VectorSubcoreMesh(**_MESH),
        out_type=jax.ShapeDtypeStruct((NC, N_PAD), jnp.float32),
        scratch_types=[
            pltpu.VMEM((NBLK, BLKI, CHUNK), jnp.int32),
            pltpu.VMEM((CHUNK,), jnp.float32),
            pltpu.VMEM_SHARED((N_PAD,), jnp.float32),
        ],
    )
    def k(dsts_hbm, ones_hbm, zeros_hbm, out_hbm, dst_v, ones_v, acc_sh):
        c = lax.axis_index("c")
        s = lax.axis_index("s")
        rows = pl.ds(s * RPT, RPT)
        pltpu.sync_copy(dsts_hbm.at[c, s], dst_v)
        pltpu.sync_copy(ones_hbm, ones_v)
        pltpu.sync_copy(zeros_hbm.at[rows], acc_sh.at[rows])
        plsc.subcore_barrier()

        def body(B, carry):
            for r in range(BLKI):
                pltpu.sync_copy(ones_v, acc_sh.at[dst_v.at[B, r]], add=True)
            return carry

        lax.fori_loop(0, NBLK, body, 0)
        plsc.subcore_barrier()
        pltpu.sync_copy(acc_sh.at[rows], out_hbm.at[c, rows])

    return k(dsts, ones1, zeros1)


def _sc_prop(u, srcs, dsts, zeros):
    @functools.partial(
        pl.kernel,
        mesh=plsc.VectorSubcoreMesh(**_MESH),
        out_type=jax.ShapeDtypeStruct((NC, N_PAD, D), jnp.float32),
        scratch_types=[
            pltpu.VMEM((2, BLKI, CHUNK), jnp.int32),
            pltpu.VMEM((2, BLKI, CHUNK), jnp.int32),
            pltpu.VMEM((2, CHUNK, D), jnp.float32),
            pltpu.VMEM_SHARED((N_PAD, D), jnp.float32),
            pltpu.SemaphoreType.DMA,
            pltpu.SemaphoreType.DMA,
            pltpu.SemaphoreType.DMA,
        ],
    )
    def k(u_hbm, srcs_hbm, dsts_hbm, z_hbm, out_hbm,
          sidx, didx, buf_v, acc_sh, sg0, sg1, si):
        c = lax.axis_index("c")
        s = lax.axis_index("s")
        rows = pl.ds(s * RPT, RPT)
        sgs = (sg0, sg1)

        pltpu.async_copy(srcs_hbm.at[c, s, 0], sidx.at[0], si)
        pltpu.async_copy(dsts_hbm.at[c, s, 0], didx.at[0], si)
        pltpu.async_copy(srcs_hbm.at[c, s, 1], sidx.at[1], si)
        pltpu.async_copy(dsts_hbm.at[c, s, 1], didx.at[1], si)

        @pl.when(c == 0)
        def _():
            pltpu.sync_copy(u_hbm.at[rows], acc_sh.at[rows])

        @pl.when(c != 0)
        def _():
            pltpu.sync_copy(z_hbm.at[rows], acc_sh.at[rows])

        pltpu.make_async_copy(srcs_hbm.at[c, s, 0], sidx.at[0], si).wait()
        pltpu.make_async_copy(dsts_hbm.at[c, s, 0], didx.at[0], si).wait()
        plsc.subcore_barrier()

        pltpu.async_copy(u_hbm.at[sidx.at[0, 0]], buf_v.at[0], sg0)
        pltpu.async_copy(u_hbm.at[sidx.at[0, 1]], buf_v.at[1], sg1)

        def block(B, pb):
            for r in range(BLKI):
                if r == BLKI - 2:
                    @pl.when(B + 1 < NBLK)
                    def _():
                        pltpu.make_async_copy(
                            srcs_hbm.at[c, s, 0], sidx.at[1 - pb], si).wait()
                        pltpu.make_async_copy(
                            dsts_hbm.at[c, s, 0], didx.at[1 - pb], si).wait()
                par = r % 2
                pltpu.make_async_copy(
                    u_hbm.at[sidx.at[pb, r]], buf_v.at[par], sgs[par]).wait()
                pltpu.sync_copy(
                    buf_v.at[par], acc_sh.at[didx.at[pb, r]], add=True)
                if r < BLKI - 2:
                    pltpu.async_copy(
                        u_hbm.at[sidx.at[pb, r + 2]], buf_v.at[par], sgs[par])
                else:
                    @pl.when(B + 1 < NBLK)
                    def _():
                        pltpu.async_copy(
                            u_hbm.at[sidx.at[1 - pb, r - (BLKI - 2)]],
                            buf_v.at[par], sgs[par])

            @pl.when(B + 2 < NBLK)
            def _():
                pltpu.async_copy(srcs_hbm.at[c, s, B + 2], sidx.at[pb], si)
                pltpu.async_copy(dsts_hbm.at[c, s, B + 2], didx.at[pb], si)

        def pair(i, carry):
            block(2 * i, 0)
            block(2 * i + 1, 1)
            return carry

        lax.fori_loop(0, NBLK // 2, pair, 0)
        plsc.subcore_barrier()
        pltpu.sync_copy(acc_sh.at[rows], out_hbm.at[c, rows])

    return k(u, srcs, dsts, zeros)


_BLK = 2048


def _dinv_of(deg_ref):
    return lax.rsqrt(deg_ref[:, 0:1] + deg_ref[:, 1:2] + 1.0)


def _tc_first(x, W, deg):
    def body(x_ref, w_ref, deg_ref, u_ref):
        dinv = _dinv_of(deg_ref)
        u_ref[...] = dinv * jnp.dot(
            x_ref[...], w_ref[...], preferred_element_type=jnp.float32)

    return pl.pallas_call(
        body,
        grid=(N_PAD // _BLK,),
        in_specs=[
            pl.BlockSpec((_BLK, D), lambda i: (i, 0)),
            pl.BlockSpec((D, D), lambda i: (0, 0)),
            pl.BlockSpec((_BLK, NC), lambda i: (i, 0)),
        ],
        out_specs=pl.BlockSpec((_BLK, D), lambda i: (i, 0)),
        out_shape=jax.ShapeDtypeStruct((N_PAD, D), jnp.float32),
    )(x, W, deg)


def _tc_mid(part, b, W, deg):
    def body(p_ref, b_ref, w_ref, deg_ref, u_ref):
        dinv = _dinv_of(deg_ref)
        t = dinv * (p_ref[0] + p_ref[1]) + b_ref[...]
        t = jnp.maximum(t, 0.0)
        u_ref[...] = dinv * jnp.dot(
            t, w_ref[...], preferred_element_type=jnp.float32)

    return pl.pallas_call(
        body,
        grid=(N_PAD // _BLK,),
        in_specs=[
            pl.BlockSpec((NC, _BLK, D), lambda i: (0, i, 0)),
            pl.BlockSpec((1, D), lambda i: (0, 0)),
            pl.BlockSpec((D, D), lambda i: (0, 0)),
            pl.BlockSpec((_BLK, NC), lambda i: (i, 0)),
        ],
        out_specs=pl.BlockSpec((_BLK, D), lambda i: (i, 0)),
        out_shape=jax.ShapeDtypeStruct((N_PAD, D), jnp.float32),
    )(part, b, W, deg)


def _tc_last(part, b, deg):
    def body(p_ref, b_ref, deg_ref, o_ref):
        dinv = _dinv_of(deg_ref)
        o_ref[...] = dinv * (p_ref[0] + p_ref[1]) + b_ref[...]

    return pl.pallas_call(
        body,
        grid=(N_PAD // _BLK,),
        in_specs=[
            pl.BlockSpec((NC, _BLK, D), lambda i: (0, i, 0)),
            pl.BlockSpec((1, D), lambda i: (0, 0)),
            pl.BlockSpec((_BLK, NC), lambda i: (i, 0)),
        ],
        out_specs=pl.BlockSpec((_BLK, D), lambda i: (i, 0)),
        out_shape=jax.ShapeDtypeStruct((N_PAD, D), jnp.float32),
    )(part, b, deg)


def kernel(x, edge_index, W1, b1, W2, b2, W3, b3):
    src = edge_index[0].astype(jnp.int32)
    dst = edge_index[1].astype(jnp.int32)
    pad = N + jnp.arange(E_PAD - E, dtype=jnp.int32) % (N_PAD - N)
    srcs = jnp.concatenate([src, pad]).reshape(NC, NS, NBLK, BLKI, CHUNK)
    dsts = jnp.concatenate([dst, pad]).reshape(NC, NS, NBLK, BLKI, CHUNK)

    x_pad = jnp.zeros((N_PAD, D), jnp.float32).at[:N].set(x)
    zeros128 = jnp.zeros((N_PAD, D), jnp.float32)
    zeros1 = jnp.zeros((N_PAD,), jnp.float32)
    ones1 = jnp.ones((CHUNK,), jnp.float32)
    b1r = b1.reshape(1, D)
    b2r = b2.reshape(1, D)
    b3r = b3.reshape(1, D)

    deg = _sc_deg(dsts, ones1, zeros1).T
    u = _tc_first(x_pad, W1, deg)
    p = _sc_prop(u, srcs, dsts, zeros128)
    u = _tc_mid(p, b1r, W2, deg)
    p = _sc_prop(u, srcs, dsts, zeros128)
    u = _tc_mid(p, b2r, W3, deg)
    p = _sc_prop(u, srcs, dsts, zeros128)
    out = _tc_last(p, b3r, deg)
    return out[:N]
